# Initial kernel scaffold; baseline (speedup 1.0000x reference)
import jax
import jax.numpy as jnp
from jax.experimental import pallas as pl


def kernel(node_feats, edge_index, edge_weights, W1, W2, gn1_gamma, gn1_beta, gn1_alpha, gn2_gamma, gn2_beta, gn2_alpha, r1_phi_w, r1_phi_b, r1_rho_w, r1_rho_b, r2_phi_w, r2_phi_b, r2_rho_w, r2_rho_b):
    raise NotImplementedError("write your pallas kernel here")


# speedup vs baseline: 4.6523x; 4.6523x over previous
import functools

import jax
import jax.numpy as jnp
from jax import lax
from jax.experimental import pallas as pl
from jax.experimental.pallas import tpu as pltpu
from jax.experimental.pallas import tpu_sc as plsc

NN = 10000
EE = 320000
HH = 128
RD = 64
NC = 2
NS = 16
NW = NC * NS
L = 16
NPAD = 10112
K = 128
NCHUNK = EE // K
DEG_C = 2000
EPT = EE // NW
RPT = NPAD // NS

_mesh = plsc.VectorSubcoreMesh(
    core_axis_name="c", subcore_axis_name="s", num_cores=NC, num_subcores=NS
)


def _leaky(x):
    return jnp.where(x > 0, x, 0.01 * x)


@functools.partial(
    pl.kernel,
    out_type=jax.ShapeDtypeStruct((NW, 2, NPAD), jnp.float32),
    mesh=_mesh,
    compiler_params=pltpu.CompilerParams(needs_layout_passes=False),
    scratch_types=[
        pltpu.VMEM((NPAD,), jnp.float32),
        pltpu.VMEM((NPAD,), jnp.float32),
        pltpu.VMEM((DEG_C,), jnp.int32),
        pltpu.VMEM((DEG_C,), jnp.int32),
    ],
)
def _deg_kernel(src_hbm, dst_hbm, out_hbm, sdeg, ddeg, sbuf, dbuf):
    cid = lax.axis_index("c")
    sid = lax.axis_index("s")
    wid = sid * NC + cid

    zeros = jnp.zeros((L,), jnp.float32)

    def zb(i, _):
        sdeg[pl.ds(i * L, L)] = zeros
        ddeg[pl.ds(i * L, L)] = zeros
        return 0

    lax.fori_loop(0, NPAD // L, zb, 0)

    ones = jnp.ones((L,), jnp.float32)
    e0 = wid * EPT

    def cb(c, _):
        off = e0 + c * DEG_C
        pltpu.sync_copy(src_hbm.at[pl.ds(off, DEG_C)], sbuf)
        pltpu.sync_copy(dst_hbm.at[pl.ds(off, DEG_C)], dbuf)

        def ib(i, _):
            s = sbuf[pl.ds(i * L, L)]
            d = dbuf[pl.ds(i * L, L)]
            plsc.addupdate_scatter(sdeg, [s], ones)
            plsc.addupdate_scatter(ddeg, [d], ones)
            return 0

        lax.fori_loop(0, DEG_C // L, ib, 0)
        return 0

    lax.fori_loop(0, EPT // DEG_C, cb, 0)

    pltpu.sync_copy(sdeg, out_hbm.at[wid, 0])
    pltpu.sync_copy(ddeg, out_hbm.at[wid, 1])


@functools.partial(
    pl.kernel,
    out_type=jax.ShapeDtypeStruct((NC, NPAD, HH), jnp.float32),
    mesh=_mesh,
    compiler_params=pltpu.CompilerParams(needs_layout_passes=False),
    scratch_types=[
        pltpu.MemorySpace.VMEM_SHARED((NPAD, HH), jnp.float32),
        pltpu.VMEM((NPAD,), jnp.float32),
        pltpu.VMEM((NPAD,), jnp.float32),
        pltpu.VMEM((K,), jnp.int32),
        pltpu.VMEM((1, K), jnp.int32),
        pltpu.VMEM((K,), jnp.float32),
        pltpu.VMEM((K,), jnp.float32),
        pltpu.VMEM((K, HH), jnp.float32),
        pltpu.SemaphoreType.DMA,
    ],
)
def _mp_kernel(h_hbm, src_hbm, dst_hbm, ew_hbm, norms_hbm, out_hbm,
               agg, ns, nd, sbuf, dbuf, wbuf, pbuf, rows, sem):
    cid = lax.axis_index("c")
    sid = lax.axis_index("s")
    wid = sid * NC + cid

    pltpu.sync_copy(norms_hbm.at[0], ns)
    pltpu.sync_copy(norms_hbm.at[1], nd)

    zeros = jnp.zeros((L,), jnp.float32)

    def zb(r, _):
        for f in range(HH // L):
            rows[r, pl.ds(f * L, L)] = zeros
        return 0

    lax.fori_loop(0, K, zb, 0)

    base = sid * RPT
    nfull = RPT // K
    rem = RPT - nfull * K
    for j in range(nfull):
        pltpu.sync_copy(rows, agg.at[pl.ds(base + j * K, K)])
    if rem:
        pltpu.sync_copy(rows.at[pl.ds(0, rem)],
                        agg.at[pl.ds(base + nfull * K, rem)])
    plsc.subcore_barrier()

    nch = (NCHUNK - 1 - wid) // NW + 1

    def chunk_body(i, _):
        off = (wid + i * NW) * K
        pltpu.sync_copy(src_hbm.at[pl.ds(off, K)], sbuf)
        pltpu.sync_copy(dst_hbm.at[pl.ds(off, K)], dbuf.at[0])
        pltpu.sync_copy(ew_hbm.at[pl.ds(off, K)], wbuf)

        pltpu.make_async_copy(h_hbm.at[sbuf], rows, sem).start()

        def pb(q, _):
            sv = sbuf[pl.ds(q * L, L)]
            dv = dbuf[0, pl.ds(q * L, L)]
            wv = wbuf[pl.ds(q * L, L)]
            nsv = plsc.load_gather(ns, [sv])
            ndv = plsc.load_gather(nd, [dv])
            pbuf[pl.ds(q * L, L)] = wv * nsv * ndv
            return 0

        lax.fori_loop(0, K // L, pb, 0)
        pltpu.make_async_copy(h_hbm.at[sbuf], rows, sem).wait()

        def rb(r, _):
            ws = plsc.load_gather(pbuf, [jnp.full((L,), r, jnp.int32)])
            for f in range(HH // L):
                rows[r, pl.ds(f * L, L)] = rows[r, pl.ds(f * L, L)] * ws
            return 0

        lax.fori_loop(0, K, rb, 0)

        cp = pltpu.make_async_copy(rows, agg.at[dbuf.at[0]], sem)
        cp.start(add=True)
        cp.wait()
        return 0

    lax.fori_loop(0, nch, chunk_body, 0)
    plsc.subcore_barrier()

    pltpu.sync_copy(agg.at[pl.ds(base, RPT)],
                    out_hbm.at[cid, pl.ds(base, RPT)])


def _norms_tc(parts_ref, out_ref):
    deg = jnp.sum(parts_ref[...], axis=0)
    out_ref[...] = lax.rsqrt(jnp.maximum(deg, 1.0))


def _layer1_tc(aggp_ref, w_ref, gamma_ref, beta_ref, alpha_ref,
               phiw_ref, phib_ref, rhow_ref, rhob_ref,
               h_ref, ro_ref, mean_ref):
    agg = aggp_ref[0, :NN, :] + aggp_ref[1, :NN, :]
    hlin = jnp.dot(agg, w_ref[...], preferred_element_type=jnp.float32)
    mean = jnp.mean(hlin, axis=0, keepdims=True)
    sub = hlin - alpha_ref[...] * mean
    var = jnp.mean(sub * sub, axis=0, keepdims=True)
    h = gamma_ref[...] * sub * lax.rsqrt(var + 1e-5) + beta_ref[...]
    h = _leaky(h)
    h_ref[...] = h
    mean_ref[...] = jnp.mean(h, axis=0, keepdims=True)
    ph = _leaky(jnp.dot(h, phiw_ref[...], preferred_element_type=jnp.float32)
                + phib_ref[...])
    s = jnp.sum(ph, axis=0, keepdims=True)
    ro = jnp.dot(s, rhow_ref[...], preferred_element_type=jnp.float32) \
        + rhob_ref[...]
    ro_ref[...] = _leaky(ro)


def _layer2_tc(aggp_ref, w_ref, gamma_ref, beta_ref, alpha_ref,
               phiw_ref, phib_ref, rhow_ref, rhob_ref,
               ro1_ref, mean1_ref, out_ref):
    agg = aggp_ref[0, :NN, :] + aggp_ref[1, :NN, :]
    hlin = jnp.dot(agg, w_ref[...], preferred_element_type=jnp.float32)
    mean = jnp.mean(hlin, axis=0, keepdims=True)
    sub = hlin - alpha_ref[...] * mean
    var = jnp.mean(sub * sub, axis=0, keepdims=True)
    h = gamma_ref[...] * sub * lax.rsqrt(var + 1e-5) + beta_ref[...]
    h = _leaky(h)
    mean2 = jnp.mean(h, axis=0, keepdims=True)
    ph = _leaky(jnp.dot(h, phiw_ref[...], preferred_element_type=jnp.float32)
                + phib_ref[...])
    s = jnp.sum(ph, axis=0, keepdims=True)
    ro2 = _leaky(jnp.dot(s, rhow_ref[...], preferred_element_type=jnp.float32)
                 + rhob_ref[...])
    full = jnp.concatenate(
        [ro1_ref[...], mean1_ref[...], ro2, mean2], axis=1)
    out_ref[...] = _leaky(full)


def kernel(node_feats, edge_index, edge_weights, W1, W2,
           gn1_gamma, gn1_beta, gn1_alpha, gn2_gamma, gn2_beta, gn2_alpha,
           r1_phi_w, r1_phi_b, r1_rho_w, r1_rho_b,
           r2_phi_w, r2_phi_b, r2_rho_w, r2_rho_b):
    src = edge_index[0]
    dst = edge_index[1]
    deg_parts = _deg_kernel(src, dst)
    norms = pl.pallas_call(
        _norms_tc,
        out_shape=jax.ShapeDtypeStruct((2, NPAD), jnp.float32),
    )(deg_parts)

    agg1p = _mp_kernel(node_feats, src, dst, edge_weights, norms)

    g1 = gn1_gamma.reshape(1, HH)
    b1 = gn1_beta.reshape(1, HH)
    a1 = gn1_alpha.reshape(1, HH)
    h1, ro1, mean1 = pl.pallas_call(
        _layer1_tc,
        out_shape=(
            jax.ShapeDtypeStruct((NN, HH), jnp.float32),
            jax.ShapeDtypeStruct((1, RD), jnp.float32),
            jax.ShapeDtypeStruct((1, HH), jnp.float32),
        ),
    )(agg1p, W1, g1, b1, a1,
      r1_phi_w, r1_phi_b.reshape(1, RD), r1_rho_w, r1_rho_b.reshape(1, RD))

    agg2p = _mp_kernel(h1, src, dst, edge_weights, norms)

    g2 = gn2_gamma.reshape(1, HH)
    b2 = gn2_beta.reshape(1, HH)
    a2 = gn2_alpha.reshape(1, HH)
    out = pl.pallas_call(
        _layer2_tc,
        out_shape=jax.ShapeDtypeStruct((1, RD + HH + RD + HH), jnp.float32),
    )(agg2p, W2, g2, b2, a2,
      r2_phi_w, r2_phi_b.reshape(1, RD), r2_rho_w, r2_rho_b.reshape(1, RD),
      ro1, mean1)
    return out


# speedup vs baseline: 10.1420x; 2.1800x over previous
import functools

import jax
import jax.numpy as jnp
from jax import lax
from jax.experimental import pallas as pl
from jax.experimental.pallas import tpu as pltpu
from jax.experimental.pallas import tpu_sc as plsc

NN = 10000
EE = 320000
HH = 128
RD = 64
NC = 2
NS = 16
NW = NC * NS
L = 16
NPAD = 10112
K = 128
NCHUNK = EE // K
DEG_C = 2000
EPT = EE // NW
RPT = NPAD // NS

_mesh = plsc.VectorSubcoreMesh(
    core_axis_name="c", subcore_axis_name="s", num_cores=NC, num_subcores=NS
)


def _leaky(x):
    return jnp.where(x > 0, x, 0.01 * x)


@functools.partial(
    pl.kernel,
    out_type=jax.ShapeDtypeStruct((NW, 2, NPAD), jnp.float32),
    mesh=_mesh,
    compiler_params=pltpu.CompilerParams(needs_layout_passes=False),
    scratch_types=[
        pltpu.VMEM((NPAD,), jnp.float32),
        pltpu.VMEM((NPAD,), jnp.float32),
        pltpu.VMEM((DEG_C,), jnp.int32),
        pltpu.VMEM((DEG_C,), jnp.int32),
    ],
)
def _deg_kernel(src_hbm, dst_hbm, out_hbm, sdeg, ddeg, sbuf, dbuf):
    cid = lax.axis_index("c")
    sid = lax.axis_index("s")
    wid = sid * NC + cid

    zeros = jnp.zeros((L,), jnp.float32)

    def zb(i, _):
        sdeg[pl.ds(i * L, L)] = zeros
        ddeg[pl.ds(i * L, L)] = zeros
        return 0

    lax.fori_loop(0, NPAD // L, zb, 0)

    ones = jnp.ones((L,), jnp.float32)
    e0 = wid * EPT

    def cb(c, _):
        off = e0 + c * DEG_C
        pltpu.sync_copy(src_hbm.at[pl.ds(off, DEG_C)], sbuf)
        pltpu.sync_copy(dst_hbm.at[pl.ds(off, DEG_C)], dbuf)

        def ib(i, _):
            s = sbuf[pl.ds(i * L, L)]
            d = dbuf[pl.ds(i * L, L)]
            plsc.addupdate_scatter(sdeg, [s], ones)
            plsc.addupdate_scatter(ddeg, [d], ones)
            return 0

        lax.fori_loop(0, DEG_C // L, ib, 0)
        return 0

    lax.fori_loop(0, EPT // DEG_C, cb, 0)

    pltpu.sync_copy(sdeg, out_hbm.at[wid, 0])
    pltpu.sync_copy(ddeg, out_hbm.at[wid, 1])


NCH0 = NCHUNK // NW


@functools.partial(
    pl.kernel,
    out_type=jax.ShapeDtypeStruct((NC, NN, HH), jnp.float32),
    mesh=_mesh,
    compiler_params=pltpu.CompilerParams(needs_layout_passes=False),
    scratch_types=[
        pltpu.MemorySpace.VMEM_SHARED((NN, HH), jnp.float32),
        pltpu.VMEM((K,), jnp.int32),
        pltpu.VMEM((K,), jnp.int32),
        pltpu.VMEM((K,), jnp.int32),
        pltpu.VMEM((K,), jnp.float32),
        pltpu.VMEM((K,), jnp.float32),
        pltpu.VMEM((K,), jnp.float32),
        pltpu.VMEM((1, K), jnp.int32),
        pltpu.VMEM((1, K), jnp.int32),
        pltpu.VMEM((1, K), jnp.int32),
        pltpu.VMEM((K, HH), jnp.float32),
        pltpu.VMEM((K, HH), jnp.float32),
        pltpu.VMEM((K, HH), jnp.float32),
        pltpu.SemaphoreType.DMA,
        pltpu.SemaphoreType.DMA,
        pltpu.SemaphoreType.DMA,
        pltpu.SemaphoreType.DMA,
        pltpu.SemaphoreType.DMA,
        pltpu.SemaphoreType.DMA,
        pltpu.SemaphoreType.DMA,
        pltpu.SemaphoreType.DMA,
        pltpu.SemaphoreType.DMA,
        pltpu.SemaphoreType.DMA,
        pltpu.SemaphoreType.DMA,
        pltpu.SemaphoreType.DMA,
    ],
)
def _mp_kernel(h_hbm, src_hbm, dst_hbm, ew_hbm, out_hbm,
               agg, sb0, sb1, sb2, eb0, eb1, eb2, db0, db1, db2,
               rows0, rows1, rows2,
               g0, g1, g2, s0, s1, s2, e0, e1, e2, d0, d1, d2):
    cid = lax.axis_index("c")
    sid = lax.axis_index("s")
    wid = sid * NC + cid
    sbs = (sb0, sb1, sb2)
    ebs = (eb0, eb1, eb2)
    dbs = (db0, db1, db2)
    rows = (rows0, rows1, rows2)
    gsem = (g0, g1, g2)
    ssem = (s0, s1, s2)
    esem = (e0, e1, e2)
    dsem = (d0, d1, d2)

    cstart = wid * NCH0 + jnp.minimum(wid, 4)
    extra = wid < 4

    zeros = jnp.zeros((L,), jnp.float32)

    def zb(r, _):
        for f in range(HH // L):
            rows0[r, pl.ds(f * L, L)] = zeros
        return 0

    lax.fori_loop(0, K, zb, 0)

    base = sid * 632
    for j in range(4):
        pltpu.sync_copy(rows0, agg.at[pl.ds(base + j * K, K)])

    @pl.when(sid < NS - 1)
    def _():
        pltpu.sync_copy(rows0.at[pl.ds(0, 120)],
                        agg.at[pl.ds(base + 4 * K, 120)])

    @pl.when(sid == NS - 1)
    def _():
        pltpu.sync_copy(rows0.at[pl.ds(0, 8)],
                        agg.at[pl.ds(base + 4 * K, 8)])

    plsc.subcore_barrier()

    def eoff(i):
        return (cstart + i) * K

    def start_se(b, i):
        off = eoff(i)
        pltpu.make_async_copy(src_hbm.at[pl.ds(off, K)], sbs[b],
                              esem[b]).start()
        pltpu.make_async_copy(ew_hbm.at[pl.ds(off, K)], ebs[b],
                              esem[b]).start()

    def wait_se(b):
        pltpu.make_async_copy(src_hbm.at[pl.ds(0, K)], sbs[b],
                              esem[b]).wait()
        pltpu.make_async_copy(ew_hbm.at[pl.ds(0, K)], ebs[b],
                              esem[b]).wait()

    def start_d(b, i):
        pltpu.make_async_copy(dst_hbm.at[pl.ds(eoff(i), K)], dbs[b].at[0],
                              dsem[b]).start()

    def wait_d(b):
        pltpu.make_async_copy(dst_hbm.at[pl.ds(0, K)], dbs[b].at[0],
                              dsem[b]).wait()

    def start_g(b):
        pltpu.make_async_copy(h_hbm.at[sbs[b]], rows[b], gsem[b]).start()

    def wait_g(b):
        pltpu.make_async_copy(h_hbm.at[sbs[b]], rows[b], gsem[b]).wait()

    def start_s(b):
        pltpu.make_async_copy(rows[b], agg.at[dbs[b].at[0]],
                              ssem[b]).start(add=True)

    def wait_s(b):
        pltpu.make_async_copy(rows[b], agg.at[dbs[b].at[0]],
                              ssem[b]).wait()

    def compute(b):
        rb_ = rows[b]
        eb_ = ebs[b]

        def rloop(r, _):
            ws = plsc.load_gather(eb_, [jnp.full((L,), r, jnp.int32)])
            for f in range(HH // L):
                rb_[r, pl.ds(f * L, L)] = rb_[r, pl.ds(f * L, L)] * ws
            return 0

        lax.fori_loop(0, K, rloop, 0, unroll=4)

    start_se(0, 0)
    start_se(1, 1)
    start_d(0, 0)
    wait_se(0)
    start_g(0)

    def piter(i3, _):
        for j in range(3):
            i = i3 * 3 + j
            nxt = (j + 1) % 3
            wait_g(j)

            @pl.when(i + 1 < NCH0)
            def _():
                @pl.when(i >= 2)
                def _():
                    wait_s(nxt)

                wait_se(nxt)
                start_g(nxt)
                start_d(nxt, i + 1)

            compute(j)
            wait_d(j)
            start_s(j)

            @pl.when(i + 2 < NCH0)
            def _():
                start_se((j + 2) % 3, i + 2)

        return 0

    lax.fori_loop(0, NCH0 // 3, piter, 0)
    wait_s(0)
    wait_s(1)
    wait_s(2)

    @pl.when(extra)
    def _():
        start_se(0, NCH0)
        start_d(0, NCH0)
        wait_se(0)
        start_g(0)
        wait_g(0)
        compute(0)
        wait_d(0)
        start_s(0)
        wait_s(0)

    plsc.subcore_barrier()

    for j in range(4):
        pltpu.sync_copy(agg.at[pl.ds(base + j * K, K)],
                        out_hbm.at[cid, pl.ds(base + j * K, K)])

    @pl.when(sid < NS - 1)
    def _():
        pltpu.sync_copy(agg.at[pl.ds(base + 4 * K, 120)],
                        out_hbm.at[cid, pl.ds(base + 4 * K, 120)])

    @pl.when(sid == NS - 1)
    def _():
        pltpu.sync_copy(agg.at[pl.ds(base + 4 * K, 8)],
                        out_hbm.at[cid, pl.ds(base + 4 * K, 8)])


def _norms_tc(parts_ref, out_ref):
    deg = jnp.sum(parts_ref[...], axis=0)
    out_ref[...] = lax.rsqrt(jnp.maximum(deg, 1.0))


def _scale0_tc(x_ref, nscol_ref, out_ref):
    out_ref[...] = x_ref[...] * nscol_ref[...]


def _layer1_tc(aggp_ref, ndcol_ref, nscol_ref, w_ref,
               gamma_ref, beta_ref, alpha_ref,
               phiw_ref, phib_ref, rhow_ref, rhob_ref,
               hs_ref, ro_ref, mean_ref):
    agg = (aggp_ref[0] + aggp_ref[1]) * ndcol_ref[...]
    hlin = jnp.dot(agg, w_ref[...], preferred_element_type=jnp.float32)
    mean = jnp.mean(hlin, axis=0, keepdims=True)
    sub = hlin - alpha_ref[...] * mean
    var = jnp.mean(sub * sub, axis=0, keepdims=True)
    h = gamma_ref[...] * sub * lax.rsqrt(var + 1e-5) + beta_ref[...]
    h = _leaky(h)
    mean_ref[...] = jnp.mean(h, axis=0, keepdims=True)
    ph = _leaky(jnp.dot(h, phiw_ref[...], preferred_element_type=jnp.float32)
                + phib_ref[...])
    s = jnp.sum(ph, axis=0, keepdims=True)
    ro = jnp.dot(s, rhow_ref[...], preferred_element_type=jnp.float32) \
        + rhob_ref[...]
    ro_ref[...] = _leaky(ro)
    hs_ref[...] = h * nscol_ref[...]


def _layer2_tc(aggp_ref, ndcol_ref, w_ref,
               gamma_ref, beta_ref, alpha_ref,
               phiw_ref, phib_ref, rhow_ref, rhob_ref,
               ro1_ref, mean1_ref, out_ref):
    agg = (aggp_ref[0] + aggp_ref[1]) * ndcol_ref[...]
    hlin = jnp.dot(agg, w_ref[...], preferred_element_type=jnp.float32)
    mean = jnp.mean(hlin, axis=0, keepdims=True)
    sub = hlin - alpha_ref[...] * mean
    var = jnp.mean(sub * sub, axis=0, keepdims=True)
    h = gamma_ref[...] * sub * lax.rsqrt(var + 1e-5) + beta_ref[...]
    h = _leaky(h)
    mean2 = jnp.mean(h, axis=0, keepdims=True)
    ph = _leaky(jnp.dot(h, phiw_ref[...], preferred_element_type=jnp.float32)
                + phib_ref[...])
    s = jnp.sum(ph, axis=0, keepdims=True)
    ro2 = _leaky(jnp.dot(s, rhow_ref[...], preferred_element_type=jnp.float32)
                 + rhob_ref[...])
    full = jnp.concatenate(
        [ro1_ref[...], mean1_ref[...], ro2, mean2], axis=1)
    out_ref[...] = _leaky(full)


def kernel(node_feats, edge_index, edge_weights, W1, W2,
           gn1_gamma, gn1_beta, gn1_alpha, gn2_gamma, gn2_beta, gn2_alpha,
           r1_phi_w, r1_phi_b, r1_rho_w, r1_rho_b,
           r2_phi_w, r2_phi_b, r2_rho_w, r2_rho_b):
    src = edge_index[0]
    dst = edge_index[1]
    deg_parts = _deg_kernel(src, dst)
    norms = pl.pallas_call(
        _norms_tc,
        out_shape=jax.ShapeDtypeStruct((2, NPAD), jnp.float32),
    )(deg_parts)

    ns_col = norms[0, :NN].reshape(NN, 1)
    nd_col = norms[1, :NN].reshape(NN, 1)

    xs = pl.pallas_call(
        _scale0_tc,
        out_shape=jax.ShapeDtypeStruct((NN, HH), jnp.float32),
    )(node_feats, ns_col)

    agg1p = _mp_kernel(xs, src, dst, edge_weights)

    g1 = gn1_gamma.reshape(1, HH)
    b1 = gn1_beta.reshape(1, HH)
    a1 = gn1_alpha.reshape(1, HH)
    h1s, ro1, mean1 = pl.pallas_call(
        _layer1_tc,
        out_shape=(
            jax.ShapeDtypeStruct((NN, HH), jnp.float32),
            jax.ShapeDtypeStruct((1, RD), jnp.float32),
            jax.ShapeDtypeStruct((1, HH), jnp.float32),
        ),
    )(agg1p, nd_col, ns_col, W1, g1, b1, a1,
      r1_phi_w, r1_phi_b.reshape(1, RD), r1_rho_w, r1_rho_b.reshape(1, RD))

    agg2p = _mp_kernel(h1s, src, dst, edge_weights)

    g2 = gn2_gamma.reshape(1, HH)
    b2 = gn2_beta.reshape(1, HH)
    a2 = gn2_alpha.reshape(1, HH)
    out = pl.pallas_call(
        _layer2_tc,
        out_shape=jax.ShapeDtypeStruct((1, RD + HH + RD + HH), jnp.float32),
    )(agg2p, nd_col, W2, g2, b2, a2,
      r2_phi_w, r2_phi_b.reshape(1, RD), r2_rho_w, r2_rho_b.reshape(1, RD),
      ro1, mean1)
    return out


# speedup vs baseline: 10.2916x; 1.0148x over previous
import functools

import jax
import jax.numpy as jnp
from jax import lax
from jax.experimental import pallas as pl
from jax.experimental.pallas import tpu as pltpu
from jax.experimental.pallas import tpu_sc as plsc

NN = 10000
EE = 320000
HH = 128
RD = 64
NC = 2
NS = 16
NW = NC * NS
L = 16
NPAD = 10112
K = 128
NCHUNK = EE // K
DEG_C = 2000
EPT = EE // NW
RPT = NPAD // NS

_mesh = plsc.VectorSubcoreMesh(
    core_axis_name="c", subcore_axis_name="s", num_cores=NC, num_subcores=NS
)


def _leaky(x):
    return jnp.where(x > 0, x, 0.01 * x)


@functools.partial(
    pl.kernel,
    out_type=jax.ShapeDtypeStruct((NW, 2, NPAD), jnp.float32),
    mesh=_mesh,
    compiler_params=pltpu.CompilerParams(needs_layout_passes=False),
    scratch_types=[
        pltpu.VMEM((NPAD,), jnp.float32),
        pltpu.VMEM((NPAD,), jnp.float32),
        pltpu.VMEM((DEG_C,), jnp.int32),
        pltpu.VMEM((DEG_C,), jnp.int32),
    ],
)
def _deg_kernel(src_hbm, dst_hbm, out_hbm, sdeg, ddeg, sbuf, dbuf):
    cid = lax.axis_index("c")
    sid = lax.axis_index("s")
    wid = sid * NC + cid

    zeros = jnp.zeros((L,), jnp.float32)

    def zb(i, _):
        sdeg[pl.ds(i * L, L)] = zeros
        ddeg[pl.ds(i * L, L)] = zeros
        return 0

    lax.fori_loop(0, NPAD // L, zb, 0)

    ones = jnp.ones((L,), jnp.float32)
    e0 = wid * EPT

    def cb(c, _):
        off = e0 + c * DEG_C
        pltpu.sync_copy(src_hbm.at[pl.ds(off, DEG_C)], sbuf)
        pltpu.sync_copy(dst_hbm.at[pl.ds(off, DEG_C)], dbuf)

        def ib(i, _):
            s = sbuf[pl.ds(i * L, L)]
            d = dbuf[pl.ds(i * L, L)]
            plsc.addupdate_scatter(sdeg, [s], ones)
            plsc.addupdate_scatter(ddeg, [d], ones)
            return 0

        lax.fori_loop(0, DEG_C // L, ib, 0)
        return 0

    lax.fori_loop(0, EPT // DEG_C, cb, 0)

    pltpu.sync_copy(sdeg, out_hbm.at[wid, 0])
    pltpu.sync_copy(ddeg, out_hbm.at[wid, 1])


NCH0 = NCHUNK // NW


@functools.partial(
    pl.kernel,
    out_type=jax.ShapeDtypeStruct((NC, NN, HH), jnp.float32),
    mesh=_mesh,
    compiler_params=pltpu.CompilerParams(needs_layout_passes=False),
    scratch_types=[
        pltpu.MemorySpace.VMEM_SHARED((NN, HH), jnp.float32),
        pltpu.VMEM((K,), jnp.int32),
        pltpu.VMEM((K,), jnp.int32),
        pltpu.VMEM((K,), jnp.int32),
        pltpu.VMEM((K,), jnp.float32),
        pltpu.VMEM((K,), jnp.float32),
        pltpu.VMEM((K,), jnp.float32),
        pltpu.VMEM((1, K), jnp.int32),
        pltpu.VMEM((1, K), jnp.int32),
        pltpu.VMEM((1, K), jnp.int32),
        pltpu.VMEM((K, HH), jnp.float32),
        pltpu.VMEM((K, HH), jnp.float32),
        pltpu.VMEM((K, HH), jnp.float32),
        pltpu.SemaphoreType.DMA,
        pltpu.SemaphoreType.DMA,
        pltpu.SemaphoreType.DMA,
        pltpu.SemaphoreType.DMA,
        pltpu.SemaphoreType.DMA,
        pltpu.SemaphoreType.DMA,
        pltpu.SemaphoreType.DMA,
        pltpu.SemaphoreType.DMA,
        pltpu.SemaphoreType.DMA,
        pltpu.SemaphoreType.DMA,
        pltpu.SemaphoreType.DMA,
        pltpu.SemaphoreType.DMA,
    ],
)
def _mp_kernel(h_hbm, src_hbm, dst_hbm, ew_hbm, out_hbm,
               agg, sb0, sb1, sb2, eb0, eb1, eb2, db0, db1, db2,
               rows0, rows1, rows2,
               g0, g1, g2, s0, s1, s2, e0, e1, e2, d0, d1, d2):
    cid = lax.axis_index("c")
    sid = lax.axis_index("s")
    wid = sid * NC + cid
    sbs = (sb0, sb1, sb2)
    ebs = (eb0, eb1, eb2)
    dbs = (db0, db1, db2)
    rows = (rows0, rows1, rows2)
    gsem = (g0, g1, g2)
    ssem = (s0, s1, s2)
    esem = (e0, e1, e2)
    dsem = (d0, d1, d2)

    cstart = wid * NCH0 + jnp.minimum(wid, 4)
    extra = wid < 4

    zeros = jnp.zeros((L,), jnp.float32)

    def zb(r, _):
        for f in range(HH // L):
            rows0[r, pl.ds(f * L, L)] = zeros
        return 0

    lax.fori_loop(0, K, zb, 0)

    base = sid * 632
    for j in range(4):
        pltpu.sync_copy(rows0, agg.at[pl.ds(base + j * K, K)])

    @pl.when(sid < NS - 1)
    def _():
        pltpu.sync_copy(rows0.at[pl.ds(0, 120)],
                        agg.at[pl.ds(base + 4 * K, 120)])

    @pl.when(sid == NS - 1)
    def _():
        pltpu.sync_copy(rows0.at[pl.ds(0, 8)],
                        agg.at[pl.ds(base + 4 * K, 8)])

    plsc.subcore_barrier()

    def eoff(i):
        return (cstart + i) * K

    def start_se(b, i):
        off = eoff(i)
        pltpu.make_async_copy(src_hbm.at[pl.ds(off, K)], sbs[b],
                              esem[b]).start()
        pltpu.make_async_copy(ew_hbm.at[pl.ds(off, K)], ebs[b],
                              esem[b]).start()

    def wait_se(b):
        pltpu.make_async_copy(src_hbm.at[pl.ds(0, K)], sbs[b],
                              esem[b]).wait()
        pltpu.make_async_copy(ew_hbm.at[pl.ds(0, K)], ebs[b],
                              esem[b]).wait()

    def start_d(b, i):
        pltpu.make_async_copy(dst_hbm.at[pl.ds(eoff(i), K)], dbs[b].at[0],
                              dsem[b]).start()

    def wait_d(b):
        pltpu.make_async_copy(dst_hbm.at[pl.ds(0, K)], dbs[b].at[0],
                              dsem[b]).wait()

    def start_g(b):
        pltpu.make_async_copy(h_hbm.at[sbs[b]], rows[b], gsem[b]).start()

    def wait_g(b):
        pltpu.make_async_copy(h_hbm.at[sbs[b]], rows[b], gsem[b]).wait()

    def start_s(b):
        pltpu.make_async_copy(rows[b], agg.at[dbs[b].at[0]],
                              ssem[b]).start(add=True)

    def wait_s(b):
        pltpu.make_async_copy(rows[b], agg.at[dbs[b].at[0]],
                              ssem[b]).wait()

    def compute(b):
        rb_ = rows[b]
        eb_ = ebs[b]

        def rloop(r, _):
            ws = plsc.load_gather(eb_, [jnp.full((L,), r, jnp.int32)])
            for f in range(HH // L):
                rb_[r, pl.ds(f * L, L)] = rb_[r, pl.ds(f * L, L)] * ws
            return 0

        lax.fori_loop(0, K, rloop, 0, unroll=4)

    start_se(0, 0)
    start_se(1, 1)
    start_d(0, 0)
    wait_se(0)
    start_g(0)

    def piter(i3, _):
        for j in range(3):
            i = i3 * 3 + j
            nxt = (j + 1) % 3
            wait_g(j)

            @pl.when(i + 1 < NCH0)
            def _():

                @pl.when(i >= 2)
                def _():
                    wait_s(nxt)

                wait_se(nxt)
                start_g(nxt)
                start_d(nxt, i + 1)

            compute(j)
            wait_d(j)
            start_s(j)

            @pl.when(i + 2 < NCH0)
            def _():
                start_se((j + 2) % 3, i + 2)

        return 0

    lax.fori_loop(0, NCH0 // 3, piter, 0)
    wait_s(0)
    wait_s(1)
    wait_s(2)

    @pl.when(extra)
    def _():
        start_se(0, NCH0)
        start_d(0, NCH0)
        wait_se(0)
        start_g(0)
        wait_g(0)
        compute(0)
        wait_d(0)
        start_s(0)
        wait_s(0)

    plsc.subcore_barrier()

    for j in range(4):
        pltpu.sync_copy(agg.at[pl.ds(base + j * K, K)],
                        out_hbm.at[cid, pl.ds(base + j * K, K)])

    @pl.when(sid < NS - 1)
    def _():
        pltpu.sync_copy(agg.at[pl.ds(base + 4 * K, 120)],
                        out_hbm.at[cid, pl.ds(base + 4 * K, 120)])

    @pl.when(sid == NS - 1)
    def _():
        pltpu.sync_copy(agg.at[pl.ds(base + 4 * K, 8)],
                        out_hbm.at[cid, pl.ds(base + 4 * K, 8)])


def _prep_tc(parts_ref, x_ref, xs_ref, nscol_ref, ndcol_ref):
    deg = jnp.sum(parts_ref[...], axis=0)
    norms = lax.rsqrt(jnp.maximum(deg, 1.0))
    norms_t = jnp.transpose(norms, (1, 0))
    nscol = norms_t[:NN, 0:1]
    ndcol = norms_t[:NN, 1:2]
    nscol_ref[...] = nscol
    ndcol_ref[...] = ndcol
    xs_ref[...] = x_ref[...] * nscol


def _layer1_tc(aggp_ref, ndcol_ref, nscol_ref, w_ref,
               gamma_ref, beta_ref, alpha_ref,
               phiw_ref, phib_ref, rhow_ref, rhob_ref,
               hs_ref, ro_ref, mean_ref):
    agg = (aggp_ref[0] + aggp_ref[1]) * ndcol_ref[...]
    hlin = jnp.dot(agg, w_ref[...], preferred_element_type=jnp.float32)
    mean = jnp.mean(hlin, axis=0, keepdims=True)
    sub = hlin - alpha_ref[...] * mean
    var = jnp.mean(sub * sub, axis=0, keepdims=True)
    h = gamma_ref[...] * sub * lax.rsqrt(var + 1e-5) + beta_ref[...]
    h = _leaky(h)
    mean_ref[...] = jnp.mean(h, axis=0, keepdims=True)
    ph = _leaky(jnp.dot(h, phiw_ref[...], preferred_element_type=jnp.float32)
                + phib_ref[...])
    s = jnp.sum(ph, axis=0, keepdims=True)
    ro = jnp.dot(s, rhow_ref[...], preferred_element_type=jnp.float32) \
        + rhob_ref[...]
    ro_ref[...] = _leaky(ro)
    hs_ref[...] = h * nscol_ref[...]


def _layer2_tc(aggp_ref, ndcol_ref, w_ref,
               gamma_ref, beta_ref, alpha_ref,
               phiw_ref, phib_ref, rhow_ref, rhob_ref,
               ro1_ref, mean1_ref, out_ref):
    agg = (aggp_ref[0] + aggp_ref[1]) * ndcol_ref[...]
    hlin = jnp.dot(agg, w_ref[...], preferred_element_type=jnp.float32)
    mean = jnp.mean(hlin, axis=0, keepdims=True)
    sub = hlin - alpha_ref[...] * mean
    var = jnp.mean(sub * sub, axis=0, keepdims=True)
    h = gamma_ref[...] * sub * lax.rsqrt(var + 1e-5) + beta_ref[...]
    h = _leaky(h)
    mean2 = jnp.mean(h, axis=0, keepdims=True)
    ph = _leaky(jnp.dot(h, phiw_ref[...], preferred_element_type=jnp.float32)
                + phib_ref[...])
    s = jnp.sum(ph, axis=0, keepdims=True)
    ro2 = _leaky(jnp.dot(s, rhow_ref[...], preferred_element_type=jnp.float32)
                 + rhob_ref[...])
    full = jnp.concatenate(
        [ro1_ref[...], mean1_ref[...], ro2, mean2], axis=1)
    out_ref[...] = _leaky(full)


def kernel(node_feats, edge_index, edge_weights, W1, W2,
           gn1_gamma, gn1_beta, gn1_alpha, gn2_gamma, gn2_beta, gn2_alpha,
           r1_phi_w, r1_phi_b, r1_rho_w, r1_rho_b,
           r2_phi_w, r2_phi_b, r2_rho_w, r2_rho_b):
    src = edge_index[0]
    dst = edge_index[1]
    deg_parts = _deg_kernel(src, dst)
    xs, ns_col, nd_col = pl.pallas_call(
        _prep_tc,
        out_shape=(
            jax.ShapeDtypeStruct((NN, HH), jnp.float32),
            jax.ShapeDtypeStruct((NN, 1), jnp.float32),
            jax.ShapeDtypeStruct((NN, 1), jnp.float32),
        ),
    )(deg_parts, node_feats)

    agg1p = _mp_kernel(xs, src, dst, edge_weights)

    g1 = gn1_gamma.reshape(1, HH)
    b1 = gn1_beta.reshape(1, HH)
    a1 = gn1_alpha.reshape(1, HH)
    h1s, ro1, mean1 = pl.pallas_call(
        _layer1_tc,
        out_shape=(
            jax.ShapeDtypeStruct((NN, HH), jnp.float32),
            jax.ShapeDtypeStruct((1, RD), jnp.float32),
            jax.ShapeDtypeStruct((1, HH), jnp.float32),
        ),
    )(agg1p, nd_col, ns_col, W1, g1, b1, a1,
      r1_phi_w, r1_phi_b.reshape(1, RD), r1_rho_w, r1_rho_b.reshape(1, RD))

    agg2p = _mp_kernel(h1s, src, dst, edge_weights)

    g2 = gn2_gamma.reshape(1, HH)
    b2 = gn2_beta.reshape(1, HH)
    a2 = gn2_alpha.reshape(1, HH)
    out = pl.pallas_call(
        _layer2_tc,
        out_shape=jax.ShapeDtypeStruct((1, RD + HH + RD + HH), jnp.float32),
    )(agg2p, nd_col, W2, g2, b2, a2,
      r2_phi_w, r2_phi_b.reshape(1, RD), r2_rho_w, r2_rho_b.reshape(1, RD),
      ro1, mean1)
    return out


# speedup vs baseline: 11.4564x; 1.1132x over previous
import functools

import jax
import jax.numpy as jnp
from jax import lax
from jax.experimental import pallas as pl
from jax.experimental.pallas import tpu as pltpu
from jax.experimental.pallas import tpu_sc as plsc

NN = 10000
EE = 320000
HH = 128
RD = 64
NC = 2
NS = 16
NW = NC * NS
L = 16
NPAD = 10112
K = 128
NCHUNK = EE // K
DEG_C = 2000
EPT = EE // NW
RPT = NPAD // NS

_mesh = plsc.VectorSubcoreMesh(
    core_axis_name="c", subcore_axis_name="s", num_cores=NC, num_subcores=NS
)


def _leaky(x):
    return jnp.where(x > 0, x, 0.01 * x)


@functools.partial(
    pl.kernel,
    out_type=jax.ShapeDtypeStruct((NW, 2, NPAD), jnp.float32),
    mesh=_mesh,
    compiler_params=pltpu.CompilerParams(needs_layout_passes=False),
    scratch_types=[
        pltpu.VMEM((NPAD,), jnp.float32),
        pltpu.VMEM((NPAD,), jnp.float32),
        pltpu.VMEM((DEG_C,), jnp.int32),
        pltpu.VMEM((DEG_C,), jnp.int32),
    ],
)
def _deg_kernel(src_hbm, dst_hbm, out_hbm, sdeg, ddeg, sbuf, dbuf):
    cid = lax.axis_index("c")
    sid = lax.axis_index("s")
    wid = sid * NC + cid

    zeros = jnp.zeros((L,), jnp.float32)

    def zb(i, _):
        sdeg[pl.ds(i * L, L)] = zeros
        ddeg[pl.ds(i * L, L)] = zeros
        return 0

    lax.fori_loop(0, NPAD // L, zb, 0)

    ones = jnp.ones((L,), jnp.float32)
    e0 = wid * EPT

    def cb(c, _):
        off = e0 + c * DEG_C
        pltpu.sync_copy(src_hbm.at[pl.ds(off, DEG_C)], sbuf)
        pltpu.sync_copy(dst_hbm.at[pl.ds(off, DEG_C)], dbuf)

        def ib(i, _):
            s = sbuf[pl.ds(i * L, L)]
            d = dbuf[pl.ds(i * L, L)]
            plsc.addupdate_scatter(sdeg, [s], ones)
            plsc.addupdate_scatter(ddeg, [d], ones)
            return 0

        lax.fori_loop(0, DEG_C // L, ib, 0)
        return 0

    lax.fori_loop(0, EPT // DEG_C, cb, 0)

    pltpu.sync_copy(sdeg, out_hbm.at[wid, 0])
    pltpu.sync_copy(ddeg, out_hbm.at[wid, 1])


NCH0 = NCHUNK // NW


@functools.partial(
    pl.kernel,
    out_type=jax.ShapeDtypeStruct((NC, NN, HH), jnp.float32),
    mesh=_mesh,
    compiler_params=pltpu.CompilerParams(needs_layout_passes=False),
    scratch_types=[
        pltpu.MemorySpace.VMEM_SHARED((NN, HH), jnp.float32),
        pltpu.VMEM((K,), jnp.int32),
        pltpu.VMEM((K,), jnp.int32),
        pltpu.VMEM((K,), jnp.int32),
        pltpu.VMEM((K,), jnp.float32),
        pltpu.VMEM((K,), jnp.float32),
        pltpu.VMEM((K,), jnp.float32),
        pltpu.VMEM((1, K), jnp.int32),
        pltpu.VMEM((1, K), jnp.int32),
        pltpu.VMEM((1, K), jnp.int32),
        pltpu.VMEM((K, HH), jnp.float32),
        pltpu.VMEM((K, HH), jnp.float32),
        pltpu.VMEM((K, HH), jnp.float32),
        pltpu.SemaphoreType.DMA,
        pltpu.SemaphoreType.DMA,
        pltpu.SemaphoreType.DMA,
        pltpu.SemaphoreType.DMA,
        pltpu.SemaphoreType.DMA,
        pltpu.SemaphoreType.DMA,
        pltpu.SemaphoreType.DMA,
        pltpu.SemaphoreType.DMA,
        pltpu.SemaphoreType.DMA,
        pltpu.SemaphoreType.DMA,
        pltpu.SemaphoreType.DMA,
        pltpu.SemaphoreType.DMA,
    ],
)
def _mp_kernel(h_hbm, src_hbm, dst_hbm, ew_hbm, out_hbm,
               agg, sb0, sb1, sb2, eb0, eb1, eb2, db0, db1, db2,
               rows0, rows1, rows2,
               g0, g1, g2, s0, s1, s2, e0, e1, e2, d0, d1, d2):
    cid = lax.axis_index("c")
    sid = lax.axis_index("s")
    wid = sid * NC + cid
    sbs = (sb0, sb1, sb2)
    ebs = (eb0, eb1, eb2)
    dbs = (db0, db1, db2)
    rows = (rows0, rows1, rows2)
    gsem = (g0, g1, g2)
    ssem = (s0, s1, s2)
    esem = (e0, e1, e2)
    dsem = (d0, d1, d2)

    cstart = wid * NCH0 + jnp.minimum(wid, 4)
    extra = wid < 4

    zeros = jnp.zeros((L,), jnp.float32)

    def zb(r, _):
        for f in range(HH // L):
            rows0[r, pl.ds(f * L, L)] = zeros
        return 0

    lax.fori_loop(0, K, zb, 0)

    base = sid * 632
    for j in range(4):
        pltpu.sync_copy(rows0, agg.at[pl.ds(base + j * K, K)])

    @pl.when(sid < NS - 1)
    def _():
        pltpu.sync_copy(rows0.at[pl.ds(0, 120)],
                        agg.at[pl.ds(base + 4 * K, 120)])

    @pl.when(sid == NS - 1)
    def _():
        pltpu.sync_copy(rows0.at[pl.ds(0, 8)],
                        agg.at[pl.ds(base + 4 * K, 8)])

    plsc.subcore_barrier()

    def eoff(i):
        return (cstart + i) * K

    def start_se(b, i):
        off = eoff(i)
        pltpu.make_async_copy(src_hbm.at[pl.ds(off, K)], sbs[b],
                              esem[b]).start()
        pltpu.make_async_copy(ew_hbm.at[pl.ds(off, K)], ebs[b],
                              esem[b]).start()

    def wait_se(b):
        pltpu.make_async_copy(src_hbm.at[pl.ds(0, K)], sbs[b],
                              esem[b]).wait()
        pltpu.make_async_copy(ew_hbm.at[pl.ds(0, K)], ebs[b],
                              esem[b]).wait()

    def start_d(b, i):
        pltpu.make_async_copy(dst_hbm.at[pl.ds(eoff(i), K)], dbs[b].at[0],
                              dsem[b]).start()

    def wait_d(b):
        pltpu.make_async_copy(dst_hbm.at[pl.ds(0, K)], dbs[b].at[0],
                              dsem[b]).wait()

    def start_g(b):
        pltpu.make_async_copy(h_hbm.at[sbs[b]], rows[b], gsem[b]).start()

    def wait_g(b):
        pltpu.make_async_copy(h_hbm.at[sbs[b]], rows[b], gsem[b]).wait()

    def start_s(b):
        pltpu.make_async_copy(rows[b], agg.at[dbs[b].at[0]],
                              ssem[b]).start(add=True)

    def wait_s(b):
        pltpu.make_async_copy(rows[b], agg.at[dbs[b].at[0]],
                              ssem[b]).wait()

    def compute(b):
        rb_ = rows[b]
        eb_ = ebs[b]

        def rloop(r, _):
            ws = plsc.load_gather(eb_, [jnp.full((L,), r, jnp.int32)])
            for f in range(HH // L):
                rb_[r, pl.ds(f * L, L)] = rb_[r, pl.ds(f * L, L)] * ws
            return 0

        lax.fori_loop(0, K, rloop, 0, unroll=4)

    start_se(0, 0)
    start_se(1, 1)
    start_se(2, 2)
    start_d(0, 0)
    start_d(1, 1)
    wait_se(0)
    start_g(0)
    wait_se(1)
    start_g(1)

    def piter(i3, _):
        for j in range(3):
            i = i3 * 3 + j
            nx2 = (j + 2) % 3
            wait_g(j)
            compute(j)
            wait_d(j)
            start_s(j)

            @pl.when(i + 2 < NCH0)
            def _():

                @pl.when(i >= 1)
                def _():
                    wait_s(nx2)

                wait_se(nx2)
                start_g(nx2)
                start_d(nx2, i + 2)

            @pl.when(i + 3 < NCH0)
            def _():
                start_se(j, i + 3)

        return 0

    lax.fori_loop(0, NCH0 // 3, piter, 0)
    wait_s(0)
    wait_s(1)
    wait_s(2)

    @pl.when(extra)
    def _():
        start_se(0, NCH0)
        start_d(0, NCH0)
        wait_se(0)
        start_g(0)
        wait_g(0)
        compute(0)
        wait_d(0)
        start_s(0)
        wait_s(0)

    plsc.subcore_barrier()

    for j in range(4):
        pltpu.sync_copy(agg.at[pl.ds(base + j * K, K)],
                        out_hbm.at[cid, pl.ds(base + j * K, K)])

    @pl.when(sid < NS - 1)
    def _():
        pltpu.sync_copy(agg.at[pl.ds(base + 4 * K, 120)],
                        out_hbm.at[cid, pl.ds(base + 4 * K, 120)])

    @pl.when(sid == NS - 1)
    def _():
        pltpu.sync_copy(agg.at[pl.ds(base + 4 * K, 8)],
                        out_hbm.at[cid, pl.ds(base + 4 * K, 8)])


def _prep_tc(parts_ref, x_ref, xs_ref, nscol_ref, ndcol_ref):
    deg = jnp.sum(parts_ref[...], axis=0)
    norms = lax.rsqrt(jnp.maximum(deg, 1.0))
    norms_t = jnp.transpose(norms, (1, 0))
    nscol = norms_t[:NN, 0:1]
    ndcol = norms_t[:NN, 1:2]
    nscol_ref[...] = nscol
    ndcol_ref[...] = ndcol
    xs_ref[...] = x_ref[...] * nscol


def _layer1_tc(aggp_ref, ndcol_ref, nscol_ref, w_ref,
               gamma_ref, beta_ref, alpha_ref,
               phiw_ref, phib_ref, rhow_ref, rhob_ref,
               hs_ref, ro_ref, mean_ref):
    agg = (aggp_ref[0] + aggp_ref[1]) * ndcol_ref[...]
    hlin = jnp.dot(agg, w_ref[...], preferred_element_type=jnp.float32)
    mean = jnp.mean(hlin, axis=0, keepdims=True)
    sub = hlin - alpha_ref[...] * mean
    var = jnp.mean(sub * sub, axis=0, keepdims=True)
    h = gamma_ref[...] * sub * lax.rsqrt(var + 1e-5) + beta_ref[...]
    h = _leaky(h)
    mean_ref[...] = jnp.mean(h, axis=0, keepdims=True)
    ph = _leaky(jnp.dot(h, phiw_ref[...], preferred_element_type=jnp.float32)
                + phib_ref[...])
    s = jnp.sum(ph, axis=0, keepdims=True)
    ro = jnp.dot(s, rhow_ref[...], preferred_element_type=jnp.float32) \
        + rhob_ref[...]
    ro_ref[...] = _leaky(ro)
    hs_ref[...] = h * nscol_ref[...]


def _layer2_tc(aggp_ref, ndcol_ref, w_ref,
               gamma_ref, beta_ref, alpha_ref,
               phiw_ref, phib_ref, rhow_ref, rhob_ref,
               ro1_ref, mean1_ref, out_ref):
    agg = (aggp_ref[0] + aggp_ref[1]) * ndcol_ref[...]
    hlin = jnp.dot(agg, w_ref[...], preferred_element_type=jnp.float32)
    mean = jnp.mean(hlin, axis=0, keepdims=True)
    sub = hlin - alpha_ref[...] * mean
    var = jnp.mean(sub * sub, axis=0, keepdims=True)
    h = gamma_ref[...] * sub * lax.rsqrt(var + 1e-5) + beta_ref[...]
    h = _leaky(h)
    mean2 = jnp.mean(h, axis=0, keepdims=True)
    ph = _leaky(jnp.dot(h, phiw_ref[...], preferred_element_type=jnp.float32)
                + phib_ref[...])
    s = jnp.sum(ph, axis=0, keepdims=True)
    ro2 = _leaky(jnp.dot(s, rhow_ref[...], preferred_element_type=jnp.float32)
                 + rhob_ref[...])
    full = jnp.concatenate(
        [ro1_ref[...], mean1_ref[...], ro2, mean2], axis=1)
    out_ref[...] = _leaky(full)


def kernel(node_feats, edge_index, edge_weights, W1, W2,
           gn1_gamma, gn1_beta, gn1_alpha, gn2_gamma, gn2_beta, gn2_alpha,
           r1_phi_w, r1_phi_b, r1_rho_w, r1_rho_b,
           r2_phi_w, r2_phi_b, r2_rho_w, r2_rho_b):
    src = edge_index[0]
    dst = edge_index[1]
    deg_parts = _deg_kernel(src, dst)
    xs, ns_col, nd_col = pl.pallas_call(
        _prep_tc,
        out_shape=(
            jax.ShapeDtypeStruct((NN, HH), jnp.float32),
            jax.ShapeDtypeStruct((NN, 1), jnp.float32),
            jax.ShapeDtypeStruct((NN, 1), jnp.float32),
        ),
    )(deg_parts, node_feats)

    agg1p = _mp_kernel(xs, src, dst, edge_weights)

    g1 = gn1_gamma.reshape(1, HH)
    b1 = gn1_beta.reshape(1, HH)
    a1 = gn1_alpha.reshape(1, HH)
    h1s, ro1, mean1 = pl.pallas_call(
        _layer1_tc,
        out_shape=(
            jax.ShapeDtypeStruct((NN, HH), jnp.float32),
            jax.ShapeDtypeStruct((1, RD), jnp.float32),
            jax.ShapeDtypeStruct((1, HH), jnp.float32),
        ),
    )(agg1p, nd_col, ns_col, W1, g1, b1, a1,
      r1_phi_w, r1_phi_b.reshape(1, RD), r1_rho_w, r1_rho_b.reshape(1, RD))

    agg2p = _mp_kernel(h1s, src, dst, edge_weights)

    g2 = gn2_gamma.reshape(1, HH)
    b2 = gn2_beta.reshape(1, HH)
    a2 = gn2_alpha.reshape(1, HH)
    out = pl.pallas_call(
        _layer2_tc,
        out_shape=jax.ShapeDtypeStruct((1, RD + HH + RD + HH), jnp.float32),
    )(agg2p, nd_col, W2, g2, b2, a2,
      r2_phi_w, r2_phi_b.reshape(1, RD), r2_rho_w, r2_rho_b.reshape(1, RD),
      ro1, mean1)
    return out


# speedup vs baseline: 11.6040x; 1.0129x over previous
import functools

import jax
import jax.numpy as jnp
from jax import lax
from jax.experimental import pallas as pl
from jax.experimental.pallas import tpu as pltpu
from jax.experimental.pallas import tpu_sc as plsc

NN = 10000
EE = 320000
HH = 128
RD = 64
NC = 2
NS = 16
NW = NC * NS
L = 16
NPAD = 10112
K = 128
NCHUNK = EE // K
DEG_C = 2000
EPT = EE // NW
RPT = NPAD // NS

_mesh = plsc.VectorSubcoreMesh(
    core_axis_name="c", subcore_axis_name="s", num_cores=NC, num_subcores=NS
)


def _leaky(x):
    return jnp.where(x > 0, x, 0.01 * x)


@functools.partial(
    pl.kernel,
    out_type=jax.ShapeDtypeStruct((NW, 2, NPAD), jnp.float32),
    mesh=_mesh,
    compiler_params=pltpu.CompilerParams(needs_layout_passes=False),
    scratch_types=[
        pltpu.VMEM((NPAD,), jnp.float32),
        pltpu.VMEM((NPAD,), jnp.float32),
        pltpu.VMEM((DEG_C,), jnp.int32),
        pltpu.VMEM((DEG_C,), jnp.int32),
        pltpu.VMEM((DEG_C,), jnp.int32),
        pltpu.VMEM((DEG_C,), jnp.int32),
        pltpu.SemaphoreType.DMA,
        pltpu.SemaphoreType.DMA,
    ],
)
def _deg_kernel(src_hbm, dst_hbm, out_hbm, sdeg, ddeg,
                sbuf0, dbuf0, sbuf1, dbuf1, em0, em1):
    cid = lax.axis_index("c")
    sid = lax.axis_index("s")
    wid = sid * NC + cid
    sbufs = (sbuf0, sbuf1)
    dbufs = (dbuf0, dbuf1)
    esems = (em0, em1)
    e0 = wid * EPT
    nch = EPT // DEG_C

    def start_e(b, c):
        off = e0 + c * DEG_C
        pltpu.make_async_copy(src_hbm.at[pl.ds(off, DEG_C)], sbufs[b],
                              esems[b]).start()
        pltpu.make_async_copy(dst_hbm.at[pl.ds(off, DEG_C)], dbufs[b],
                              esems[b]).start()

    def wait_e(b):
        pltpu.make_async_copy(src_hbm.at[pl.ds(0, DEG_C)], sbufs[b],
                              esems[b]).wait()
        pltpu.make_async_copy(dst_hbm.at[pl.ds(0, DEG_C)], dbufs[b],
                              esems[b]).wait()

    start_e(0, 0)
    start_e(1, 1)

    zeros = jnp.zeros((L,), jnp.float32)

    def zb(i, _):
        sdeg[pl.ds(i * L, L)] = zeros
        ddeg[pl.ds(i * L, L)] = zeros
        return 0

    lax.fori_loop(0, NPAD // L, zb, 0)

    ones = jnp.ones((L,), jnp.float32)

    def cb(c2, _):
        for b in range(2):
            c = c2 * 2 + b
            wait_e(b)

            def ib(i, _):
                s = sbufs[b][pl.ds(i * L, L)]
                d = dbufs[b][pl.ds(i * L, L)]
                plsc.addupdate_scatter(sdeg, [s], ones)
                plsc.addupdate_scatter(ddeg, [d], ones)
                return 0

            lax.fori_loop(0, DEG_C // L, ib, 0)

            @pl.when(c + 2 < nch)
            def _():
                start_e(b, c + 2)

        return 0

    lax.fori_loop(0, nch // 2, cb, 0)

    wait_e(0)

    def ib_last(i, _):
        s = sbuf0[pl.ds(i * L, L)]
        d = dbuf0[pl.ds(i * L, L)]
        plsc.addupdate_scatter(sdeg, [s], ones)
        plsc.addupdate_scatter(ddeg, [d], ones)
        return 0

    lax.fori_loop(0, DEG_C // L, ib_last, 0)

    cp0 = pltpu.make_async_copy(sdeg, out_hbm.at[wid, 0], em0)
    cp1 = pltpu.make_async_copy(ddeg, out_hbm.at[wid, 1], em1)
    cp0.start()
    cp1.start()
    cp0.wait()
    cp1.wait()


NCH0 = NCHUNK // NW


@functools.partial(
    pl.kernel,
    out_type=jax.ShapeDtypeStruct((NC, NN, HH), jnp.float32),
    mesh=_mesh,
    compiler_params=pltpu.CompilerParams(needs_layout_passes=False),
    scratch_types=[
        pltpu.MemorySpace.VMEM_SHARED((NN, HH), jnp.float32),
        pltpu.VMEM((K,), jnp.int32),
        pltpu.VMEM((K,), jnp.int32),
        pltpu.VMEM((K,), jnp.int32),
        pltpu.VMEM((K,), jnp.float32),
        pltpu.VMEM((K,), jnp.float32),
        pltpu.VMEM((K,), jnp.float32),
        pltpu.VMEM((1, K), jnp.int32),
        pltpu.VMEM((1, K), jnp.int32),
        pltpu.VMEM((1, K), jnp.int32),
        pltpu.VMEM((K, HH), jnp.float32),
        pltpu.VMEM((K, HH), jnp.float32),
        pltpu.VMEM((K, HH), jnp.float32),
        pltpu.SemaphoreType.DMA,
        pltpu.SemaphoreType.DMA,
        pltpu.SemaphoreType.DMA,
        pltpu.SemaphoreType.DMA,
        pltpu.SemaphoreType.DMA,
        pltpu.SemaphoreType.DMA,
        pltpu.SemaphoreType.DMA,
        pltpu.SemaphoreType.DMA,
        pltpu.SemaphoreType.DMA,
        pltpu.SemaphoreType.DMA,
        pltpu.SemaphoreType.DMA,
        pltpu.SemaphoreType.DMA,
    ],
)
def _mp_kernel(h_hbm, src_hbm, dst_hbm, ew_hbm, out_hbm,
               agg, sb0, sb1, sb2, eb0, eb1, eb2, db0, db1, db2,
               rows0, rows1, rows2,
               g0, g1, g2, s0, s1, s2, e0, e1, e2, d0, d1, d2):
    cid = lax.axis_index("c")
    sid = lax.axis_index("s")
    wid = sid * NC + cid
    sbs = (sb0, sb1, sb2)
    ebs = (eb0, eb1, eb2)
    dbs = (db0, db1, db2)
    rows = (rows0, rows1, rows2)
    gsem = (g0, g1, g2)
    ssem = (s0, s1, s2)
    esem = (e0, e1, e2)
    dsem = (d0, d1, d2)

    cstart = wid * NCH0 + jnp.minimum(wid, 4)
    extra = wid < 4

    zeros = jnp.zeros((L,), jnp.float32)

    def zb(r, _):
        for f in range(HH // L):
            rows0[r, pl.ds(f * L, L)] = zeros
        return 0

    lax.fori_loop(0, K, zb, 0)

    base = sid * 632
    zcps = [pltpu.make_async_copy(rows0, agg.at[pl.ds(base + j * K, K)], g0)
            for j in range(4)]
    for cp in zcps:
        cp.start()

    @pl.when(sid < NS - 1)
    def _():
        cp = pltpu.make_async_copy(rows0.at[pl.ds(0, 120)],
                                   agg.at[pl.ds(base + 4 * K, 120)], g1)
        cp.start()
        cp.wait()

    @pl.when(sid == NS - 1)
    def _():
        cp = pltpu.make_async_copy(rows0.at[pl.ds(0, 8)],
                                   agg.at[pl.ds(base + 4 * K, 8)], g1)
        cp.start()
        cp.wait()

    for cp in zcps:
        cp.wait()
    plsc.subcore_barrier()

    def eoff(i):
        return (cstart + i) * K

    def start_se(b, i):
        off = eoff(i)
        pltpu.make_async_copy(src_hbm.at[pl.ds(off, K)], sbs[b],
                              esem[b]).start()
        pltpu.make_async_copy(ew_hbm.at[pl.ds(off, K)], ebs[b],
                              esem[b]).start()

    def wait_se(b):
        pltpu.make_async_copy(src_hbm.at[pl.ds(0, K)], sbs[b],
                              esem[b]).wait()
        pltpu.make_async_copy(ew_hbm.at[pl.ds(0, K)], ebs[b],
                              esem[b]).wait()

    def start_d(b, i):
        pltpu.make_async_copy(dst_hbm.at[pl.ds(eoff(i), K)], dbs[b].at[0],
                              dsem[b]).start()

    def wait_d(b):
        pltpu.make_async_copy(dst_hbm.at[pl.ds(0, K)], dbs[b].at[0],
                              dsem[b]).wait()

    def start_g(b):
        pltpu.make_async_copy(h_hbm.at[sbs[b]], rows[b], gsem[b]).start()

    def wait_g(b):
        pltpu.make_async_copy(h_hbm.at[sbs[b]], rows[b], gsem[b]).wait()

    def start_s(b):
        pltpu.make_async_copy(rows[b], agg.at[dbs[b].at[0]],
                              ssem[b]).start(add=True)

    def wait_s(b):
        pltpu.make_async_copy(rows[b], agg.at[dbs[b].at[0]],
                              ssem[b]).wait()

    def compute(b):
        rb_ = rows[b]
        eb_ = ebs[b]

        def rloop(r, _):
            ws = plsc.load_gather(eb_, [jnp.full((L,), r, jnp.int32)])
            for f in range(HH // L):
                rb_[r, pl.ds(f * L, L)] = rb_[r, pl.ds(f * L, L)] * ws
            return 0

        lax.fori_loop(0, K, rloop, 0, unroll=4)

    start_se(0, 0)
    start_se(1, 1)
    start_se(2, 2)
    start_d(0, 0)
    start_d(1, 1)
    wait_se(0)
    start_g(0)
    wait_se(1)
    start_g(1)

    def piter(i3, _):
        for j in range(3):
            i = i3 * 3 + j
            nx2 = (j + 2) % 3
            wait_g(j)
            compute(j)
            wait_d(j)
            start_s(j)

            @pl.when(i + 2 < NCH0)
            def _():

                @pl.when(i >= 1)
                def _():
                    wait_s(nx2)

                wait_se(nx2)
                start_g(nx2)
                start_d(nx2, i + 2)

            @pl.when(i + 3 < NCH0)
            def _():
                start_se(j, i + 3)

        return 0

    lax.fori_loop(0, NCH0 // 3, piter, 0)
    wait_s(0)
    wait_s(1)
    wait_s(2)

    @pl.when(extra)
    def _():
        start_se(0, NCH0)
        start_d(0, NCH0)
        wait_se(0)
        start_g(0)
        wait_g(0)
        compute(0)
        wait_d(0)
        start_s(0)
        wait_s(0)

    plsc.subcore_barrier()

    dcps = [pltpu.make_async_copy(agg.at[pl.ds(base + j * K, K)],
                                  out_hbm.at[cid, pl.ds(base + j * K, K)],
                                  gsem[j % 3])
            for j in range(4)]
    for cp in dcps:
        cp.start()

    @pl.when(sid < NS - 1)
    def _():
        cp = pltpu.make_async_copy(agg.at[pl.ds(base + 4 * K, 120)],
                                   out_hbm.at[cid, pl.ds(base + 4 * K, 120)],
                                   s0)
        cp.start()
        cp.wait()

    @pl.when(sid == NS - 1)
    def _():
        cp = pltpu.make_async_copy(agg.at[pl.ds(base + 4 * K, 8)],
                                   out_hbm.at[cid, pl.ds(base + 4 * K, 8)],
                                   s0)
        cp.start()
        cp.wait()

    for cp in dcps:
        cp.wait()


def _prep_tc(parts_ref, x_ref, xs_ref, nscol_ref, ndcol_ref):
    deg = jnp.sum(parts_ref[...], axis=0)
    norms = lax.rsqrt(jnp.maximum(deg, 1.0))
    norms_t = jnp.transpose(norms, (1, 0))
    nscol = norms_t[:NN, 0:1]
    ndcol = norms_t[:NN, 1:2]
    nscol_ref[...] = nscol
    ndcol_ref[...] = ndcol
    xs_ref[...] = x_ref[...] * nscol


def _layer1_tc(aggp_ref, ndcol_ref, nscol_ref, w_ref,
               gamma_ref, beta_ref, alpha_ref,
               phiw_ref, phib_ref, rhow_ref, rhob_ref,
               hs_ref, ro_ref, mean_ref):
    agg = (aggp_ref[0] + aggp_ref[1]) * ndcol_ref[...]
    hlin = jnp.dot(agg, w_ref[...], preferred_element_type=jnp.float32)
    mean = jnp.mean(hlin, axis=0, keepdims=True)
    sub = hlin - alpha_ref[...] * mean
    var = jnp.mean(sub * sub, axis=0, keepdims=True)
    h = gamma_ref[...] * sub * lax.rsqrt(var + 1e-5) + beta_ref[...]
    h = _leaky(h)
    mean_ref[...] = jnp.mean(h, axis=0, keepdims=True)
    ph = _leaky(jnp.dot(h, phiw_ref[...], preferred_element_type=jnp.float32)
                + phib_ref[...])
    s = jnp.sum(ph, axis=0, keepdims=True)
    ro = jnp.dot(s, rhow_ref[...], preferred_element_type=jnp.float32) \
        + rhob_ref[...]
    ro_ref[...] = _leaky(ro)
    hs_ref[...] = h * nscol_ref[...]


def _layer2_tc(aggp_ref, ndcol_ref, w_ref,
               gamma_ref, beta_ref, alpha_ref,
               phiw_ref, phib_ref, rhow_ref, rhob_ref,
               ro1_ref, mean1_ref, out_ref):
    agg = (aggp_ref[0] + aggp_ref[1]) * ndcol_ref[...]
    hlin = jnp.dot(agg, w_ref[...], preferred_element_type=jnp.float32)
    mean = jnp.mean(hlin, axis=0, keepdims=True)
    sub = hlin - alpha_ref[...] * mean
    var = jnp.mean(sub * sub, axis=0, keepdims=True)
    h = gamma_ref[...] * sub * lax.rsqrt(var + 1e-5) + beta_ref[...]
    h = _leaky(h)
    mean2 = jnp.mean(h, axis=0, keepdims=True)
    ph = _leaky(jnp.dot(h, phiw_ref[...], preferred_element_type=jnp.float32)
                + phib_ref[...])
    s = jnp.sum(ph, axis=0, keepdims=True)
    ro2 = _leaky(jnp.dot(s, rhow_ref[...], preferred_element_type=jnp.float32)
                 + rhob_ref[...])
    full = jnp.concatenate(
        [ro1_ref[...], mean1_ref[...], ro2, mean2], axis=1)
    out_ref[...] = _leaky(full)


def kernel(node_feats, edge_index, edge_weights, W1, W2,
           gn1_gamma, gn1_beta, gn1_alpha, gn2_gamma, gn2_beta, gn2_alpha,
           r1_phi_w, r1_phi_b, r1_rho_w, r1_rho_b,
           r2_phi_w, r2_phi_b, r2_rho_w, r2_rho_b):
    src = edge_index[0]
    dst = edge_index[1]
    deg_parts = _deg_kernel(src, dst)
    xs, ns_col, nd_col = pl.pallas_call(
        _prep_tc,
        out_shape=(
            jax.ShapeDtypeStruct((NN, HH), jnp.float32),
            jax.ShapeDtypeStruct((NN, 1), jnp.float32),
            jax.ShapeDtypeStruct((NN, 1), jnp.float32),
        ),
    )(deg_parts, node_feats)

    agg1p = _mp_kernel(xs, src, dst, edge_weights)

    g1 = gn1_gamma.reshape(1, HH)
    b1 = gn1_beta.reshape(1, HH)
    a1 = gn1_alpha.reshape(1, HH)
    h1s, ro1, mean1 = pl.pallas_call(
        _layer1_tc,
        out_shape=(
            jax.ShapeDtypeStruct((NN, HH), jnp.float32),
            jax.ShapeDtypeStruct((1, RD), jnp.float32),
            jax.ShapeDtypeStruct((1, HH), jnp.float32),
        ),
    )(agg1p, nd_col, ns_col, W1, g1, b1, a1,
      r1_phi_w, r1_phi_b.reshape(1, RD), r1_rho_w, r1_rho_b.reshape(1, RD))

    agg2p = _mp_kernel(h1s, src, dst, edge_weights)

    g2 = gn2_gamma.reshape(1, HH)
    b2 = gn2_beta.reshape(1, HH)
    a2 = gn2_alpha.reshape(1, HH)
    out = pl.pallas_call(
        _layer2_tc,
        out_shape=jax.ShapeDtypeStruct((1, RD + HH + RD + HH), jnp.float32),
    )(agg2p, nd_col, W2, g2, b2, a2,
      r2_phi_w, r2_phi_b.reshape(1, RD), r2_rho_w, r2_rho_b.reshape(1, RD),
      ro1, mean1)
    return out
